# baseline (speedup 1.0000x reference)
import functools

import jax
import jax.numpy as jnp
from jax import lax
from jax.experimental import pallas as pl
from jax.experimental.pallas import tpu as pltpu
from jax.experimental.pallas import tpu_sc as plsc

NUM_CORES = 2
NUM_SUBCORES = 16
NUM_WORKERS = NUM_CORES * NUM_SUBCORES
BATCH = 16384
ROWS_PER_WORKER = BATCH // NUM_WORKERS
CHUNK = 32
NUM_CHUNKS = ROWS_PER_WORKER // CHUNK
MUT_LEN = 50
MAP_DIM = 64
CMD_DIM = 128
MUT_DIM = 96
AI_DIM = 32
OUT_DIM = MAP_DIM + 2 * CMD_DIM + MUT_DIM + AI_DIM
IDS_PAD = CHUNK * MUT_LEN + 16
IDS_BYTES = (CHUNK + 2 * CHUNK + CHUNK + CHUNK * MUT_LEN) * 4
OUT_BYTES = CHUNK * OUT_DIM * 4


def _sc_body(map_ids_h, cmd_ids_h, mut_ids_h, ai_ids_h,
             map_t_h, cmd_t_h, mut_t_h, ai_t_h,
             map_o_h, cmde_o_h, cmdo_o_h, mut_o_h, ai_o_h,
             mut_tab_v, mut_ids_v, mut_out_v,
             map_idx_v, map_rows_v,
             cmd_idx_v, even_v, odd_v, cmde_rows_v, cmdo_rows_v,
             ai_idx_v, ai_rows_v,
             sem_in, sem_g, sem_out):
  wid = lax.axis_index("s") * NUM_CORES + lax.axis_index("c")
  base = wid * ROWS_PER_WORKER

  def fire_ids(k, p):
    r = base + k * CHUNK
    pltpu.async_copy(map_ids_h.at[pl.ds(r, CHUNK)],
                     map_idx_v.at[pl.ds(p * CHUNK, CHUNK)], sem_in)
    pltpu.async_copy(cmd_ids_h.at[pl.ds(2 * r, 2 * CHUNK)],
                     cmd_idx_v.at[pl.ds(p * 2 * CHUNK, 2 * CHUNK)], sem_in)
    pltpu.async_copy(ai_ids_h.at[pl.ds(r, CHUNK)],
                     ai_idx_v.at[pl.ds(p * CHUNK, CHUNK)], sem_in)
    pltpu.async_copy(mut_ids_h.at[pl.ds(r * MUT_LEN, CHUNK * MUT_LEN)],
                     mut_ids_v.at[pl.ds(p * IDS_PAD, CHUNK * MUT_LEN)],
                     sem_in)

  def wait_ids():
    pltpu.make_async_copy(map_ids_h.at[pl.ds(0, CHUNK)],
                          map_idx_v.at[pl.ds(0, CHUNK)], sem_in).wait()
    pltpu.make_async_copy(cmd_ids_h.at[pl.ds(0, 2 * CHUNK)],
                          cmd_idx_v.at[pl.ds(0, 2 * CHUNK)], sem_in).wait()
    pltpu.make_async_copy(ai_ids_h.at[pl.ds(0, CHUNK)],
                          ai_idx_v.at[pl.ds(0, CHUNK)], sem_in).wait()
    pltpu.make_async_copy(mut_ids_h.at[pl.ds(0, CHUNK * MUT_LEN)],
                          mut_ids_v.at[pl.ds(0, CHUNK * MUT_LEN)],
                          sem_in).wait()

  def out_views(k, p):
    r = base + k * CHUNK
    rp = p * CHUNK
    rows = pl.ds(r, CHUNK)
    rowsp = pl.ds(rp, CHUNK)
    yield (map_rows_v.at[rowsp], map_o_h.at[rows])
    yield (cmde_rows_v.at[rowsp], cmde_o_h.at[rows])
    yield (cmdo_rows_v.at[rowsp], cmdo_o_h.at[rows])
    yield (mut_out_v.at[rowsp], mut_o_h.at[rows])
    yield (ai_rows_v.at[rowsp], ai_o_h.at[rows])

  pltpu.sync_copy(mut_t_h, mut_tab_v)

  fire_ids(0, 0)

  @pl.loop(0, NUM_CHUNKS)
  def _(k):
    p = lax.rem(k, 2)

    wait_ids()

    @pl.when(k >= 2)
    def _():
      for src, dst in out_views(k, p):
        pltpu.make_async_copy(src, dst, sem_out).wait()

    lane = lax.iota(jnp.int32, 16)
    gidx = (lane % 8) * 2
    bc = p * 2 * CHUNK
    for t in range(CHUNK // 16):
      v0 = cmd_idx_v[pl.ds(bc + 32 * t, 16)]
      v1 = cmd_idx_v[pl.ds(bc + 32 * t + 16, 16)]
      lo_e = v0.at[gidx].get(mode="promise_in_bounds")
      hi_e = v1.at[gidx].get(mode="promise_in_bounds")
      lo_o = v0.at[gidx + 1].get(mode="promise_in_bounds")
      hi_o = v1.at[gidx + 1].get(mode="promise_in_bounds")
      even_v[pl.ds(p * CHUNK + 16 * t, 16)] = jnp.where(lane < 8, lo_e, hi_e)
      odd_v[pl.ds(p * CHUNK + 16 * t, 16)] = jnp.where(lane < 8, lo_o, hi_o)

    rp = p * CHUNK
    cp_map = pltpu.async_copy(map_t_h.at[map_idx_v.at[pl.ds(rp, CHUNK)]],
                              map_rows_v.at[pl.ds(rp, CHUNK)], sem_g)
    cp_cmde = pltpu.async_copy(cmd_t_h.at[even_v.at[pl.ds(rp, CHUNK)]],
                               cmde_rows_v.at[pl.ds(rp, CHUNK)], sem_g)
    cp_cmdo = pltpu.async_copy(cmd_t_h.at[odd_v.at[pl.ds(rp, CHUNK)]],
                               cmdo_rows_v.at[pl.ds(rp, CHUNK)], sem_g)
    cp_ai = pltpu.async_copy(ai_t_h.at[ai_idx_v.at[pl.ds(rp, CHUNK)]],
                             ai_rows_v.at[pl.ds(rp, CHUNK)], sem_g)

    fire_ids(jnp.minimum(k + 1, NUM_CHUNKS - 1), 1 - p)

    bi = p * IDS_PAD
    @pl.loop(0, CHUNK)
    def _(s):
      sbase = bi + s * MUT_LEN

      def acc16(offv, accs, nlanes):
        for l in range(nlanes):
          off = pl.multiple_of(offv[l], 32)
          accs = tuple(accs[g] + mut_tab_v[pl.ds(off + 32 * g, 32)]
                       for g in range(3))
        return accs

      def tbody(t, accs):
        offv = mut_ids_v[pl.ds(sbase + 16 * t, 16)] * MUT_DIM
        return acc16(offv, accs, 16)

      accs = lax.fori_loop(0, 3, tbody,
                           tuple(jnp.zeros((32,), jnp.bfloat16)
                                 for _ in range(3)))
      tail_offv = mut_ids_v[pl.ds(sbase + 48, 16)] * MUT_DIM
      accs = acc16(tail_offv, accs, MUT_LEN - 48)
      scale = jnp.float32(1.0 / MUT_LEN)
      for g in range(3):
        lo, hi = plsc.unpack(accs[g], format=plsc.PackFormat.INTERLEAVED)
        mut_out_v[rp + s, pl.ds(32 * g, 16)] = lo * scale
        mut_out_v[rp + s, pl.ds(32 * g + 16, 16)] = hi * scale

    cp_map.wait()
    cp_cmde.wait()
    cp_cmdo.wait()
    cp_ai.wait()

    for src, dst in out_views(k, p):
      pltpu.async_copy(src, dst, sem_out)

  wait_ids()
  for kk in (NUM_CHUNKS - 2, NUM_CHUNKS - 1):
    for src, dst in out_views(kk, lax.rem(kk, 2)):
      pltpu.make_async_copy(src, dst, sem_out).wait()


@jax.jit
def _embed(map_ids, cmd_ids_flat, mut_ids_flat, ai_ids,
           map_table, cmd_table, mut_table_perm, ai_table):
  mesh = plsc.VectorSubcoreMesh(core_axis_name="c", subcore_axis_name="s",
                                num_cores=NUM_CORES,
                                num_subcores=NUM_SUBCORES)
  run = functools.partial(
      pl.kernel,
      out_type=[jax.ShapeDtypeStruct((BATCH, MAP_DIM), jnp.float32),
                jax.ShapeDtypeStruct((BATCH, CMD_DIM), jnp.float32),
                jax.ShapeDtypeStruct((BATCH, CMD_DIM), jnp.float32),
                jax.ShapeDtypeStruct((BATCH, MUT_DIM), jnp.float32),
                jax.ShapeDtypeStruct((BATCH, AI_DIM), jnp.float32)],
      mesh=mesh,
      compiler_params=pltpu.CompilerParams(use_tc_tiling_on_sc=False,
                                           needs_layout_passes=False),
      scratch_types=[
          pltpu.VMEM((1000 * MUT_DIM,), jnp.bfloat16),
          pltpu.VMEM((2 * IDS_PAD,), jnp.int32),
          pltpu.VMEM((2 * CHUNK, MUT_DIM), jnp.float32),
          pltpu.VMEM((2 * CHUNK,), jnp.int32),
          pltpu.VMEM((2 * CHUNK, MAP_DIM), jnp.float32),
          pltpu.VMEM((4 * CHUNK,), jnp.int32),
          pltpu.VMEM((2 * CHUNK,), jnp.int32),
          pltpu.VMEM((2 * CHUNK,), jnp.int32),
          pltpu.VMEM((2 * CHUNK, CMD_DIM), jnp.float32),
          pltpu.VMEM((2 * CHUNK, CMD_DIM), jnp.float32),
          pltpu.VMEM((2 * CHUNK,), jnp.int32),
          pltpu.VMEM((2 * CHUNK, AI_DIM), jnp.float32),
          pltpu.SemaphoreType.DMA,
          pltpu.SemaphoreType.DMA,
          pltpu.SemaphoreType.DMA,
      ],
  )(_sc_body)
  o_map, o_cmde, o_cmdo, o_mut, o_ai = run(
      map_ids, cmd_ids_flat, mut_ids_flat, ai_ids,
      map_table, cmd_table, mut_table_perm, ai_table)
  return jnp.concatenate([o_map, o_cmde, o_cmdo, o_mut, o_ai], axis=1)


def kernel(map_ids, commander_ids, mutation_ids, ai_ids,
           map_table, commander_table, mutation_table, ai_table):
  mut_perm = (mutation_table.reshape(-1, 3, 2, 16)
              .transpose(0, 1, 3, 2)
              .astype(jnp.bfloat16)
              .reshape(-1))
  return _embed(map_ids, commander_ids.reshape(-1), mutation_ids.reshape(-1),
                ai_ids, map_table, commander_table,
                mut_perm, ai_table)


# speedup vs baseline: 1.1341x; 1.1341x over previous
import functools

import jax
import jax.numpy as jnp
from jax import lax
from jax.experimental import pallas as pl
from jax.experimental.pallas import tpu as pltpu
from jax.experimental.pallas import tpu_sc as plsc

NUM_CORES = 2
NUM_SUBCORES = 16
NUM_WORKERS = NUM_CORES * NUM_SUBCORES
BATCH = 16384
ROWS_PER_WORKER = BATCH // NUM_WORKERS
CHUNK = 32
NUM_CHUNKS = ROWS_PER_WORKER // CHUNK
MUT_LEN = 50
MAP_DIM = 64
CMD_DIM = 128
MUT_DIM = 96
AI_DIM = 32
OUT_DIM = MAP_DIM + 2 * CMD_DIM + MUT_DIM + AI_DIM
IDS_PAD = CHUNK * MUT_LEN + 16
IDS_BYTES = (CHUNK + 2 * CHUNK + CHUNK + CHUNK * MUT_LEN) * 4
OUT_BYTES = CHUNK * OUT_DIM * 4


def _sc_body(map_ids_h, cmd_ids_h, mut_ids_h, ai_ids_h,
             map_t_h, cmd_t_h, mut_t_h, ai_t_h, out_h,
             mut_tab_v, mut_ids_v, mut_out_v,
             map_idx_v, map_rows_v,
             cmd_idx_v, even_v, odd_v, cmde_rows_v, cmdo_rows_v,
             ai_idx_v, ai_rows_v,
             sem_in, sem_g, sem_out):
  wid = lax.axis_index("s") * NUM_CORES + lax.axis_index("c")
  base = wid * ROWS_PER_WORKER

  def fire_ids(k, p):
    r = base + k * CHUNK
    pltpu.async_copy(map_ids_h.at[pl.ds(r, CHUNK)],
                     map_idx_v.at[pl.ds(p * CHUNK, CHUNK)], sem_in)
    pltpu.async_copy(cmd_ids_h.at[pl.ds(2 * r, 2 * CHUNK)],
                     cmd_idx_v.at[pl.ds(p * 2 * CHUNK, 2 * CHUNK)], sem_in)
    pltpu.async_copy(ai_ids_h.at[pl.ds(r, CHUNK)],
                     ai_idx_v.at[pl.ds(p * CHUNK, CHUNK)], sem_in)
    pltpu.async_copy(mut_ids_h.at[pl.ds(r * MUT_LEN, CHUNK * MUT_LEN)],
                     mut_ids_v.at[pl.ds(p * IDS_PAD, CHUNK * MUT_LEN)],
                     sem_in)

  def wait_ids():
    pltpu.make_async_copy(map_ids_h.at[pl.ds(0, CHUNK)],
                          map_idx_v.at[pl.ds(0, CHUNK)], sem_in).wait()
    pltpu.make_async_copy(cmd_ids_h.at[pl.ds(0, 2 * CHUNK)],
                          cmd_idx_v.at[pl.ds(0, 2 * CHUNK)], sem_in).wait()
    pltpu.make_async_copy(ai_ids_h.at[pl.ds(0, CHUNK)],
                          ai_idx_v.at[pl.ds(0, CHUNK)], sem_in).wait()
    pltpu.make_async_copy(mut_ids_h.at[pl.ds(0, CHUNK * MUT_LEN)],
                          mut_ids_v.at[pl.ds(0, CHUNK * MUT_LEN)],
                          sem_in).wait()

  def out_views(k, p):
    r = base + k * CHUNK
    rp = p * CHUNK
    yield (map_rows_v.at[pl.ds(rp, CHUNK)],
           out_h.at[pl.ds(r, CHUNK), pl.ds(0, MAP_DIM)])
    yield (cmde_rows_v.at[pl.ds(rp, CHUNK)],
           out_h.at[pl.ds(r, CHUNK), pl.ds(MAP_DIM, CMD_DIM)])
    yield (cmdo_rows_v.at[pl.ds(rp, CHUNK)],
           out_h.at[pl.ds(r, CHUNK), pl.ds(MAP_DIM + CMD_DIM, CMD_DIM)])
    yield (mut_out_v.at[pl.ds(rp, CHUNK)],
           out_h.at[pl.ds(r, CHUNK), pl.ds(MAP_DIM + 2 * CMD_DIM, MUT_DIM)])
    yield (ai_rows_v.at[pl.ds(rp, CHUNK)],
           out_h.at[pl.ds(r, CHUNK), pl.ds(OUT_DIM - AI_DIM, AI_DIM)])

  pltpu.sync_copy(mut_t_h, mut_tab_v)

  fire_ids(0, 0)

  @pl.loop(0, NUM_CHUNKS)
  def _(k):
    p = lax.rem(k, 2)

    wait_ids()

    @pl.when(k >= 2)
    def _():
      for src, dst in out_views(k, p):
        pltpu.make_async_copy(src, dst, sem_out).wait()

    lane = lax.iota(jnp.int32, 16)
    gidx = (lane % 8) * 2
    bc = p * 2 * CHUNK
    for t in range(CHUNK // 16):
      v0 = cmd_idx_v[pl.ds(bc + 32 * t, 16)]
      v1 = cmd_idx_v[pl.ds(bc + 32 * t + 16, 16)]
      lo_e = v0.at[gidx].get(mode="promise_in_bounds")
      hi_e = v1.at[gidx].get(mode="promise_in_bounds")
      lo_o = v0.at[gidx + 1].get(mode="promise_in_bounds")
      hi_o = v1.at[gidx + 1].get(mode="promise_in_bounds")
      even_v[pl.ds(p * CHUNK + 16 * t, 16)] = jnp.where(lane < 8, lo_e, hi_e)
      odd_v[pl.ds(p * CHUNK + 16 * t, 16)] = jnp.where(lane < 8, lo_o, hi_o)

    rp = p * CHUNK
    cp_map = pltpu.async_copy(map_t_h.at[map_idx_v.at[pl.ds(rp, CHUNK)]],
                              map_rows_v.at[pl.ds(rp, CHUNK)], sem_g)
    cp_cmde = pltpu.async_copy(cmd_t_h.at[even_v.at[pl.ds(rp, CHUNK)]],
                               cmde_rows_v.at[pl.ds(rp, CHUNK)], sem_g)
    cp_cmdo = pltpu.async_copy(cmd_t_h.at[odd_v.at[pl.ds(rp, CHUNK)]],
                               cmdo_rows_v.at[pl.ds(rp, CHUNK)], sem_g)
    cp_ai = pltpu.async_copy(ai_t_h.at[ai_idx_v.at[pl.ds(rp, CHUNK)]],
                             ai_rows_v.at[pl.ds(rp, CHUNK)], sem_g)

    fire_ids(jnp.minimum(k + 1, NUM_CHUNKS - 1), 1 - p)

    bi = p * IDS_PAD
    @pl.loop(0, CHUNK)
    def _(s):
      sbase = bi + s * MUT_LEN

      def acc16(offv, accs, nlanes):
        for l in range(nlanes):
          off = pl.multiple_of(offv[l], 32)
          accs = tuple(accs[g] + mut_tab_v[pl.ds(off + 32 * g, 32)]
                       for g in range(3))
        return accs

      def tbody(t, accs):
        offv = mut_ids_v[pl.ds(sbase + 16 * t, 16)] * MUT_DIM
        return acc16(offv, accs, 16)

      accs = lax.fori_loop(0, 3, tbody,
                           tuple(jnp.zeros((32,), jnp.bfloat16)
                                 for _ in range(3)))
      tail_offv = mut_ids_v[pl.ds(sbase + 48, 16)] * MUT_DIM
      accs = acc16(tail_offv, accs, MUT_LEN - 48)
      scale = jnp.float32(1.0 / MUT_LEN)
      for g in range(3):
        lo, hi = plsc.unpack(accs[g], format=plsc.PackFormat.INTERLEAVED)
        mut_out_v[rp + s, pl.ds(32 * g, 16)] = lo * scale
        mut_out_v[rp + s, pl.ds(32 * g + 16, 16)] = hi * scale

    cp_map.wait()
    cp_cmde.wait()
    cp_cmdo.wait()
    cp_ai.wait()

    for src, dst in out_views(k, p):
      pltpu.async_copy(src, dst, sem_out)

  wait_ids()
  for kk in (NUM_CHUNKS - 2, NUM_CHUNKS - 1):
    for src, dst in out_views(kk, lax.rem(kk, 2)):
      pltpu.make_async_copy(src, dst, sem_out).wait()


@jax.jit
def _embed(map_ids, cmd_ids_flat, mut_ids_flat, ai_ids,
           map_table, cmd_table, mut_table_perm, ai_table):
  mesh = plsc.VectorSubcoreMesh(core_axis_name="c", subcore_axis_name="s",
                                num_cores=NUM_CORES,
                                num_subcores=NUM_SUBCORES)
  run = functools.partial(
      pl.kernel,
      out_type=jax.ShapeDtypeStruct((BATCH, OUT_DIM), jnp.float32),
      mesh=mesh,
      compiler_params=pltpu.CompilerParams(use_tc_tiling_on_sc=False,
                                           needs_layout_passes=False),
      scratch_types=[
          pltpu.VMEM((1000 * MUT_DIM,), jnp.bfloat16),
          pltpu.VMEM((2 * IDS_PAD,), jnp.int32),
          pltpu.VMEM((2 * CHUNK, MUT_DIM), jnp.float32),
          pltpu.VMEM((2 * CHUNK,), jnp.int32),
          pltpu.VMEM((2 * CHUNK, MAP_DIM), jnp.float32),
          pltpu.VMEM((4 * CHUNK,), jnp.int32),
          pltpu.VMEM((2 * CHUNK,), jnp.int32),
          pltpu.VMEM((2 * CHUNK,), jnp.int32),
          pltpu.VMEM((2 * CHUNK, CMD_DIM), jnp.float32),
          pltpu.VMEM((2 * CHUNK, CMD_DIM), jnp.float32),
          pltpu.VMEM((2 * CHUNK,), jnp.int32),
          pltpu.VMEM((2 * CHUNK, AI_DIM), jnp.float32),
          pltpu.SemaphoreType.DMA,
          pltpu.SemaphoreType.DMA,
          pltpu.SemaphoreType.DMA,
      ],
  )(_sc_body)
  return run(map_ids, cmd_ids_flat, mut_ids_flat, ai_ids,
             map_table, cmd_table, mut_table_perm, ai_table)


def kernel(map_ids, commander_ids, mutation_ids, ai_ids,
           map_table, commander_table, mutation_table, ai_table):
  mut_perm = (mutation_table.reshape(-1, 3, 2, 16)
              .transpose(0, 1, 3, 2)
              .astype(jnp.bfloat16)
              .reshape(-1))
  return _embed(map_ids, commander_ids.reshape(-1), mutation_ids.reshape(-1),
                ai_ids, map_table, commander_table,
                mut_perm, ai_table)
